# baseline (speedup 1.0000x reference)
import functools

import jax
import jax.numpy as jnp
from jax import lax
from jax.experimental import pallas as pl
from jax.experimental.pallas import tpu as pltpu
from jax.experimental.pallas import tpu_sc as plsc

_BLK = 4096


def _zsq_tree(z):
    z2 = z * z
    c = ((z2[:, 0:8] + z2[:, 8:16]) + z2[:, 16:24]) + z2[:, 24:32]
    t1 = c[:, 0:4] + c[:, 4:8]
    t2 = t1[:, 0:2] + t1[:, 2:4]
    return t2[:, 0:1] + t2[:, 1:2]


def _vq_block(z_ref, w_ref, idx_ref, loss_ref):
    z = z_ref[...]
    w = w_ref[...]
    st = jax.lax.dot_general(
        w, z, dimension_numbers=(((1,), (1,)), ((), ())),
        preferred_element_type=jnp.float32)
    zsq = _zsq_tree(z).T
    wsq = jnp.sum(w * w, axis=1, keepdims=True)
    d2 = (zsq + wsq) - 2.0 * st
    dist = jnp.sqrt(jnp.maximum(d2, 0.0))
    m = jnp.min(dist, axis=0, keepdims=True)
    iota = jax.lax.broadcasted_iota(jnp.int32, dist.shape, 0)
    k = dist.shape[0]
    cand = jnp.where(dist == m, iota, k)
    idx = jnp.min(cand, axis=0)
    idx_ref[...] = idx[None, None, :]
    part = jnp.sum(m * m)[None, None]
    prev = jnp.where(pl.program_id(0) == 0, jnp.zeros((1, 1), jnp.float32),
                     loss_ref[...])
    loss_ref[...] = prev + part


def _tc_stage(z_e, W):
    n, d = z_e.shape
    nk = W.shape[0]
    grid = n // _BLK
    idx, loss_sum = pl.pallas_call(
        _vq_block,
        grid=(grid,),
        in_specs=[
            pl.BlockSpec((_BLK, d), lambda i: (i, 0)),
            pl.BlockSpec((nk, d), lambda i: (0, 0)),
        ],
        out_specs=[
            pl.BlockSpec((1, 1, _BLK), lambda i: (i, 0, 0)),
            pl.BlockSpec((1, 1), lambda i: (0, 0)),
        ],
        out_shape=[
            jax.ShapeDtypeStruct((grid, 1, _BLK), jnp.int32),
            jax.ShapeDtypeStruct((1, 1), jnp.float32),
        ],
    )(z_e, W)
    return idx.reshape(n), loss_sum


def _make_sc_gather(n, nk, d):
    info = plsc.get_sparse_core_info()
    nw = info.num_cores * info.num_subcores
    b_per_w = n // nw
    chunk = 1024
    n_ch = b_per_w // chunk
    mesh = plsc.VectorSubcoreMesh(core_axis_name="c", subcore_axis_name="s")

    @functools.partial(
        pl.kernel, mesh=mesh,
        compiler_params=pltpu.CompilerParams(use_tc_tiling_on_sc=False),
        out_type=jax.ShapeDtypeStruct((n, d), jnp.float32),
        scratch_types=[
            pltpu.VMEM((chunk,), jnp.int32),
            pltpu.VMEM((chunk,), jnp.int32),
            pltpu.VMEM((chunk, d), jnp.float32),
            pltpu.VMEM((chunk, d), jnp.float32),
            pltpu.SemaphoreType.DMA,
            pltpu.SemaphoreType.DMA,
            pltpu.SemaphoreType.DMA,
            pltpu.SemaphoreType.DMA,
        ],
    )
    def gather_kernel(table_hbm, idx_hbm, out_hbm,
                      idx_v0, idx_v1, rows_v0, rows_v1,
                      sem_g0, sem_g1, sem_o0, sem_o1):
        wid = lax.axis_index("s") * info.num_cores + lax.axis_index("c")
        base = wid * b_per_w
        idx_v = [idx_v0, idx_v1]
        rows_v = [rows_v0, rows_v1]
        sem_g = [sem_g0, sem_g1]
        sem_o = [sem_o0, sem_o1]
        g = [None, None]
        out_h = [None, None]
        pltpu.sync_copy(idx_hbm.at[pl.ds(base, chunk)], idx_v[0])
        g[0] = pltpu.async_copy(table_hbm.at[idx_v[0]], rows_v[0], sem_g[0])
        for c in range(n_ch):
            b = c % 2
            if c + 1 < n_ch:
                nb = (c + 1) % 2
                off_n = base + (c + 1) * chunk
                pltpu.sync_copy(idx_hbm.at[pl.ds(off_n, chunk)], idx_v[nb])
                if c + 1 >= 2:
                    out_h[nb].wait()
                g[nb] = pltpu.async_copy(
                    table_hbm.at[idx_v[nb]], rows_v[nb], sem_g[nb])
            g[b].wait()
            off = base + c * chunk
            out_h[b] = pltpu.async_copy(
                rows_v[b], out_hbm.at[pl.ds(off, chunk)], sem_o[b])
        out_h[(n_ch - 1) % 2].wait()
        if n_ch >= 2:
            out_h[n_ch % 2].wait()

    return gather_kernel


def kernel(z_e, W):
    n, d = z_e.shape
    nk = W.shape[0]
    idx, loss_sum = _tc_stage(z_e, W)
    q = _make_sc_gather(n, nk, d)(W, idx)
    vq_loss = (1.25 * loss_sum[0, 0]) / (n * d)
    return q, vq_loss


# speedup vs baseline: 1.0066x; 1.0066x over previous
import functools

import jax
import jax.numpy as jnp
from jax import lax
from jax.experimental import pallas as pl
from jax.experimental.pallas import tpu as pltpu
from jax.experimental.pallas import tpu_sc as plsc

_BLK = 4096


def _zsq_tree(z):
    z2 = z * z
    c = ((z2[:, 0:8] + z2[:, 8:16]) + z2[:, 16:24]) + z2[:, 24:32]
    t1 = c[:, 0:4] + c[:, 4:8]
    t2 = t1[:, 0:2] + t1[:, 2:4]
    return t2[:, 0:1] + t2[:, 1:2]


def _vq_block(z_ref, w_ref, idx_ref, loss_ref):
    z = z_ref[...]
    w = w_ref[...]
    st = jax.lax.dot_general(
        w, z, dimension_numbers=(((1,), (1,)), ((), ())),
        preferred_element_type=jnp.float32)
    zsq = _zsq_tree(z).T
    wsq = jnp.sum(w * w, axis=1, keepdims=True)
    d2 = (zsq + wsq) - 2.0 * st
    dist = jnp.sqrt(jnp.maximum(d2, 0.0))
    m = jnp.min(dist, axis=0, keepdims=True)
    iota = jax.lax.broadcasted_iota(jnp.int32, dist.shape, 0)
    k = dist.shape[0]
    cand = jnp.where(dist == m, iota, k)
    idx = jnp.min(cand, axis=0)
    idx_ref[...] = idx[None, None, :]
    part = jnp.sum(m * m)[None, None]
    prev = jnp.where(pl.program_id(0) == 0, jnp.zeros((1, 1), jnp.float32),
                     loss_ref[...])
    loss_ref[...] = prev + part


def _tc_stage(z_e, W):
    n, d = z_e.shape
    nk = W.shape[0]
    grid = n // _BLK
    idx, loss_sum = pl.pallas_call(
        _vq_block,
        grid=(grid,),
        in_specs=[
            pl.BlockSpec((_BLK, d), lambda i: (i, 0)),
            pl.BlockSpec((nk, d), lambda i: (0, 0)),
        ],
        out_specs=[
            pl.BlockSpec((1, 1, _BLK), lambda i: (i, 0, 0)),
            pl.BlockSpec((1, 1), lambda i: (0, 0)),
        ],
        out_shape=[
            jax.ShapeDtypeStruct((grid, 1, _BLK), jnp.int32),
            jax.ShapeDtypeStruct((1, 1), jnp.float32),
        ],
    )(z_e, W)
    return idx.reshape(n), loss_sum


def _make_sc_gather(n, nk, d):
    info = plsc.get_sparse_core_info()
    nw = info.num_cores * info.num_subcores
    b_per_w = n // nw
    chunk = 2048
    n_ch = b_per_w // chunk
    mesh = plsc.VectorSubcoreMesh(core_axis_name="c", subcore_axis_name="s")

    @functools.partial(
        pl.kernel, mesh=mesh,
        compiler_params=pltpu.CompilerParams(use_tc_tiling_on_sc=False),
        out_type=jax.ShapeDtypeStruct((n, d), jnp.float32),
        scratch_types=[
            pltpu.VMEM((chunk,), jnp.int32),
            pltpu.VMEM((chunk, d), jnp.float32),
            pltpu.SemaphoreType.DMA,
        ],
    )
    def gather_kernel(table_hbm, idx_hbm, out_hbm, idx_v, rows_v, sem):
        wid = lax.axis_index("s") * info.num_cores + lax.axis_index("c")
        base = wid * b_per_w
        for c in range(n_ch):
            off = base + c * chunk
            pltpu.sync_copy(idx_hbm.at[pl.ds(off, chunk)], idx_v)
            pltpu.async_copy(table_hbm.at[idx_v], rows_v, sem).wait()
            pltpu.sync_copy(rows_v, out_hbm.at[pl.ds(off, chunk)])

    return gather_kernel


def kernel(z_e, W):
    n, d = z_e.shape
    nk = W.shape[0]
    idx, loss_sum = _tc_stage(z_e, W)
    q = _make_sc_gather(n, nk, d)(W, idx)
    vq_loss = (1.25 * loss_sum[0, 0]) / (n * d)
    return q, vq_loss
